# Initial kernel scaffold; baseline (speedup 1.0000x reference)
import jax
import jax.numpy as jnp
from jax.experimental import pallas as pl


def kernel(latent, keys, vals):
    raise NotImplementedError("write your pallas kernel here")


# speedup vs baseline: 1.9332x; 1.9332x over previous
import jax
import jax.numpy as jnp
from jax.experimental import pallas as pl
from jax.experimental.pallas import tpu as pltpu

NQ = 1024
D = 64
NKV = 100000
BK = 2000


def _attn_kernel(lat_ref, k_ref, v_ref, o_ref, q_ref, l_ref):
    i = pl.program_id(0)
    nb = pl.num_programs(0)

    @pl.when(i == 0)
    def _init():
        lat = lat_ref[:]
        n = jnp.sqrt(jnp.sum(lat * lat, axis=1, keepdims=True))
        q_ref[:] = lat / jnp.maximum(n, 1e-12)
        l_ref[:] = jnp.zeros_like(l_ref)
        o_ref[:] = jnp.zeros_like(o_ref)

    k = k_ref[:]
    q = q_ref[:]
    sq = jax.lax.dot_general(
        jnp.ones((1, D), jnp.float32), k * k,
        (((1,), (1,)), ((), ())), preferred_element_type=jnp.float32)
    inv = jax.lax.rsqrt(jnp.maximum(sq, 1e-24))
    raw = jax.lax.dot_general(
        q, k, (((1,), (1,)), ((), ())),
        preferred_element_type=jnp.float32)
    e = jnp.exp(raw * inv)
    l_ref[:] += jnp.sum(e, axis=1, keepdims=True)
    o_ref[:] += jnp.dot(e, v_ref[:], preferred_element_type=jnp.float32)

    @pl.when(i == nb - 1)
    def _finish():
        o_ref[:] = o_ref[:] / l_ref[:]


def kernel(latent, keys, vals):
    nb = NKV // BK
    return pl.pallas_call(
        _attn_kernel,
        grid=(nb,),
        in_specs=[
            pl.BlockSpec((NQ, D), lambda i: (0, 0)),
            pl.BlockSpec((BK, D), lambda i: (i, 0)),
            pl.BlockSpec((BK, D), lambda i: (i, 0)),
        ],
        out_specs=pl.BlockSpec((NQ, D), lambda i: (0, 0)),
        out_shape=jax.ShapeDtypeStruct((NQ, D), jnp.float32),
        scratch_shapes=[
            pltpu.VMEM((NQ, D), jnp.float32),
            pltpu.VMEM((NQ, 1), jnp.float32),
        ],
    )(latent, keys, vals)


# speedup vs baseline: 1.9407x; 1.0038x over previous
import jax
import jax.numpy as jnp
from jax.experimental import pallas as pl
from jax.experimental.pallas import tpu as pltpu

NQ = 1024
D = 64
NKV = 100000
BK = 2000


def _attn_kernel(lat_ref, k_ref, v_ref, o_ref, q_ref, l_ref):
    i = pl.program_id(0)
    nb = pl.num_programs(0)

    @pl.when(i == 0)
    def _init():
        lat = lat_ref[:]
        n = jnp.sqrt(jnp.sum(lat * lat, axis=1, keepdims=True))
        q_ref[:] = (lat / jnp.maximum(n, 1e-12)).astype(jnp.bfloat16)
        l_ref[:] = jnp.zeros_like(l_ref)
        o_ref[:] = jnp.zeros_like(o_ref)

    k = k_ref[:]
    q = q_ref[:]
    sq = jax.lax.dot_general(
        jnp.ones((1, D), jnp.float32), k * k,
        (((1,), (1,)), ((), ())), preferred_element_type=jnp.float32)
    inv = jax.lax.rsqrt(jnp.maximum(sq, 1e-24)) * 1.4426950408889634
    raw = jax.lax.dot_general(
        q, k.astype(jnp.bfloat16), (((1,), (1,)), ((), ())),
        preferred_element_type=jnp.float32)
    e = jnp.exp2(raw * inv)
    l_ref[:] += jnp.sum(e, axis=1, keepdims=True)
    o_ref[:] += jnp.dot(e.astype(jnp.bfloat16), v_ref[:].astype(jnp.bfloat16),
                        preferred_element_type=jnp.float32)

    @pl.when(i == nb - 1)
    def _finish():
        o_ref[:] = o_ref[:] / l_ref[:]


def kernel(latent, keys, vals):
    nb = NKV // BK
    return pl.pallas_call(
        _attn_kernel,
        grid=(nb,),
        in_specs=[
            pl.BlockSpec((NQ, D), lambda i: (0, 0)),
            pl.BlockSpec((BK, D), lambda i: (i, 0)),
            pl.BlockSpec((BK, D), lambda i: (i, 0)),
        ],
        out_specs=pl.BlockSpec((NQ, D), lambda i: (0, 0)),
        out_shape=jax.ShapeDtypeStruct((NQ, D), jnp.float32),
        scratch_shapes=[
            pltpu.VMEM((NQ, D), jnp.bfloat16),
            pltpu.VMEM((NQ, 1), jnp.float32),
        ],
    )(latent, keys, vals)
